# baseline (speedup 1.0000x reference)
import functools

import numpy as np
import jax
import jax.numpy as jnp
from jax import lax
from jax.experimental import pallas as pl
from jax.experimental.pallas import tpu as pltpu
from jax.experimental.pallas import tpu_sc as plsc

B, Q, C = 32, 2048, 80
N = Q * C
K = 100
NBP = 17
D = 3 * NBP
CH = 8192
NCH = N // CH
GV = 32
CAP = 2048
NVCAP = CAP // 16
WPAD = 112
OPAD = 128
FLAT = OPAD * D
ROWS3 = Q * D // 128
MINI = np.int32(-2147483648)

_PERM = np.zeros(D, np.int32)
_SSEL = np.zeros(D, np.float32)
for _t in range(NBP):
    _PERM[3 * _t] = 2 * _t
    _PERM[3 * _t + 1] = 2 * _t + 1
    _PERM[3 * _t + 2] = 2 * NBP + _t
    _SSEL[3 * _t] = 0.0
    _SSEL[3 * _t + 1] = 1.0
    _SSEL[3 * _t + 2] = 2.0
_PERM_ROW = np.repeat(np.arange(OPAD, dtype=np.int32), D)
_PERM_COL = np.tile(_PERM, OPAD)
_SSEL_F = np.tile(_SSEL, OPAD)


def _skey(x):
    b = lax.bitcast_convert_type(x, jnp.int32)
    return b ^ lax.shift_right_logical(lax.shift_right_arithmetic(b, 31), 1)


def _fval(k):
    b = k ^ lax.shift_right_logical(lax.shift_right_arithmetic(k, 31), 1)
    return lax.bitcast_convert_type(b, jnp.float32)


def _popc(m):
    return jnp.sum(jnp.where(m, 1, 0))


def _sc_body(logits, kp, ts, permr, permc, ssel,
             scores_o, labels_o, kpo_o,
             chunkA, chunkB, keybuf, idxbuf, summ, wkey, widx,
             scores_st, labels_st, qidx_st, idx2a, idx2b, rows2a, rows2b, kpo_st,
             permr_v, permc_v, ssel_v, ts_v,
             semA, semB, semG):
    b = lax.axis_index("c") * 16 + lax.axis_index("s")
    iota = lax.iota(jnp.int32, 16)
    zeros16 = jnp.zeros((16,), jnp.int32)

    def count_ge(cand_s):
        cb = jnp.full((16,), cand_s, jnp.int32)

        def step(k, acc):
            uk = keybuf[pl.ds(k * 16, 16)]
            return acc + jnp.where(uk >= cb, 1, 0)

        return jnp.sum(lax.fori_loop(0, NVCAP, step, zeros16))

    def descent(limit):
        def cond(st):
            bit, _, c = st
            return (bit >= 0) & (c > limit)

        def body(st):
            bit, tb, c = st
            cand_b = tb | lax.shift_left(jnp.int32(1), bit)
            c2 = count_ge(cand_b ^ MINI)
            ok = c2 >= K
            return bit - 1, jnp.where(ok, cand_b, tb), jnp.where(ok, c2, c)

        _, tb, c = lax.while_loop(cond, body, (jnp.int32(31), jnp.int32(0), jnp.int32(2**30)))
        return tb ^ MINI, c

    def compact(cnt):
        t_s, nc0 = descent(jnp.int32(112))
        tb = jnp.full((16,), t_s, jnp.int32)

        def step(k, nc):
            uk = keybuf[pl.ds(k * 16, 16)]
            ii = idxbuf[pl.ds(k * 16, 16)]
            m = uk >= tb
            plsc.store_compressed(keybuf.at[pl.ds(nc, 16)], uk, mask=m)
            plsc.store_compressed(idxbuf.at[pl.ds(nc, 16)], ii, mask=m)
            return nc + _popc(m)

        nc = lax.fori_loop(0, NVCAP, step, jnp.int32(0))
        nc = jnp.minimum(nc, jnp.int32(CAP - 640))

        def zstep(k, _):
            pos = jnp.full((16,), k * 16, jnp.int32) + iota
            uk = keybuf[pl.ds(k * 16, 16)]
            keybuf[pl.ds(k * 16, 16)] = jnp.where(pos >= nc, MINI, uk)
            return 0

        lax.fori_loop(0, NVCAP, zstep, 0)
        return nc, jnp.max(_fval(jnp.full((16,), t_s, jnp.int32)))

    def process(chunk, cbase, carry, g0=0):
        _, thr_in = carry
        thv1 = jnp.full((16,), thr_in, jnp.float32)

        @plsc.parallel_loop(g0, CH // (GV * 16), unroll=1)
        def _pass1(g):
            xs = [chunk[pl.ds(g * (GV * 16) + v * 16, 16)] for v in range(GV)]
            while len(xs) > 1:
                xs = [jnp.maximum(a, b2) for a, b2 in zip(xs[::2], xs[1::2])]
            summ[pl.ds(g * 16, 16)] = jnp.where(xs[0] >= thv1, 1, 0)

        def group(g, st):
            def slow(st2):
                cnt2, thr2 = st2
                thv2 = jnp.full((16,), thr2, jnp.float32)
                for v in range(GV):
                    x = chunk[pl.ds(g * (GV * 16) + v * 16, 16)]
                    m = (x >= thv2) & jnp.full((16,), cnt2 <= CAP - 16, jnp.bool_)
                    idxv = jnp.full((16,), cbase + g * (GV * 16) + v * 16, jnp.int32) + iota
                    plsc.store_compressed(keybuf.at[pl.ds(cnt2, 16)], _skey(x), mask=m)
                    plsc.store_compressed(idxbuf.at[pl.ds(cnt2, 16)], idxv, mask=m)
                    cnt2 = cnt2 + _popc(m)
                return lax.cond(cnt2 > CAP - 640, lambda c: compact(c), lambda c: (c, thr2), cnt2)

            sm = summ[pl.ds(g * 16, 16)]
            return lax.cond(jnp.any(sm > 0), slow, lambda st2: st2, st)

        return lax.fori_loop(g0, CH // (GV * 16), group, carry)

    scope = jax.named_scope
    row = logits.at[b]
    pltpu.make_async_copy(row.at[pl.ds(0, CH)], chunkA, semA).start()

    def init_kb(k, _):
        keybuf[pl.ds(k * 16, 16)] = jnp.full((16,), MINI, jnp.int32)
        return 0

    with scope("ph_init"):
        lax.fori_loop(0, NVCAP, init_kb, 0)
    for s in range(OPAD // 16):
        qidx_st[pl.ds(s * 16, 16)] = zeros16
    pltpu.sync_copy(ts.at[b], ts_v)
    pltpu.sync_copy(permr, permr_v)
    pltpu.sync_copy(permc, permc_v)
    pltpu.sync_copy(ssel, ssel_v)

    with scope("ph_seed"):
        pltpu.make_async_copy(row.at[pl.ds(0, CH)], chunkA, semA).wait()

        def seed(k, _):
            keybuf[pl.ds(k * 16, 16)] = _skey(chunkA[pl.ds(k * 16, 16)])
            idxbuf[pl.ds(k * 16, 16)] = jnp.full((16,), k * 16, jnp.int32) + iota
            return 0

        lax.fori_loop(0, NVCAP, seed, 0)
        carry = compact(jnp.int32(CAP))

    pltpu.make_async_copy(row.at[pl.ds(CH, CH)], chunkB, semB).start()
    with scope("ph_filter"):
        carry = process(chunkA, 0, carry, g0=CAP // (GV * 16))

        def dbl(i, c):
            cB = 2 * i + 1
            with scope("ph_dmaw"):
                pltpu.make_async_copy(row.at[pl.ds(cB * CH, CH)], chunkB, semB).wait()

                @pl.when(cB < NCH - 1)
                def _():
                    pltpu.make_async_copy(row.at[pl.ds((cB + 1) * CH, CH)], chunkA, semA).start()

            with scope("ph_proc"):
                c = process(chunkB, cB * CH, c)

            with scope("ph_dmaw"):
                @pl.when(cB < NCH - 1)
                def _():
                    pltpu.make_async_copy(row.at[pl.ds((cB + 1) * CH, CH)], chunkA, semA).wait()

                @pl.when(i < NCH // 2 - 1)
                def _():
                    pltpu.make_async_copy(row.at[pl.ds((cB + 2) * CH, CH)], chunkB, semB).start()

            with scope("ph_proc"):
                return lax.cond(cB < NCH - 1,
                                lambda c2: process(chunkA, (cB + 1) * CH, c2),
                                lambda c2: c2, c)

        carry = lax.fori_loop(0, NCH // 2, dbl, carry)

    sctx = scope("ph_select")
    sctx.__enter__()
    t_s, _ = descent(jnp.int32(K))
    tb = jnp.full((16,), t_s, jnp.int32)

    def gcount(k, acc):
        uk = keybuf[pl.ds(k * 16, 16)]
        return acc + jnp.where(uk > tb, 1, 0)

    g = jnp.sum(lax.fori_loop(0, NVCAP, gcount, zeros16))
    r = jnp.int32(K) - g

    for s in range(WPAD // 16):
        wkey[pl.ds(s * 16, 16)] = jnp.full((16,), MINI, jnp.int32)
        widx[pl.ds(s * 16, 16)] = jnp.full((16,), jnp.int32(2**31 - 1), jnp.int32)

    def sel(k, st):
        wc, e = st
        uk = keybuf[pl.ds(k * 16, 16)]
        ii = idxbuf[pl.ds(k * 16, 16)]
        m_gt = uk > tb
        m_eq = uk == tb
        pref = plsc.cumsum(jnp.where(m_eq, 1, 0))
        keep_eq = m_eq & ((pref + e) <= r)
        m = m_gt | keep_eq
        plsc.store_compressed(wkey.at[pl.ds(wc, 16)], uk, mask=m)
        plsc.store_compressed(widx.at[pl.ds(wc, 16)], ii, mask=m)
        return wc + _popc(m), e + _popc(m_eq)

    lax.fori_loop(0, NVCAP, sel, (jnp.int32(0), jnp.int32(0)))
    sctx.__exit__(None, None, None)

    tsrow = ts_v[...]
    hsp = jnp.full((16,), jnp.sum(jnp.where(iota == 0, tsrow, 0.0)), jnp.float32)
    wsp = jnp.full((16,), jnp.sum(jnp.where(iota == 1, tsrow, 0.0)), jnp.float32)
    lane0 = iota == 0

    def rank1(i, _):
        isp = jnp.full((16,), i, jnp.int32)
        ki = plsc.load_gather(wkey, [isp])
        ni = plsc.load_gather(widx, [isp])

        def racc(a, acc):
            wk = wkey[pl.ds(a * 16, 16)]
            wi = widx[pl.ds(a * 16, 16)]
            better = (wk > ki) | ((wk == ki) & (wi < ni))
            return acc + jnp.where(better, 1, 0)

        rank = jnp.sum(lax.fori_loop(0, WPAD // 16, racc, zeros16))
        rsp = jnp.full((16,), rank, jnp.int32)
        score = 1.0 / (1.0 + jnp.exp(-_fval(ki)))
        plsc.store_scatter(scores_st, [rsp], score, mask=lane0)
        plsc.store_scatter(labels_st, [rsp], lax.rem(ni, jnp.full((16,), C, jnp.int32)), mask=lane0)
        plsc.store_scatter(qidx_st, [rsp], lax.div(ni, jnp.full((16,), C, jnp.int32)), mask=lane0)
        return 0

    with scope("ph_rank"):
        lax.fori_loop(0, K, rank1, 0)

    def pairidx(j, _):
        s = pl.ds(j * 16, 16)
        q = qidx_st[s]
        r0 = lax.shift_right_logical(q * D, 7)
        idx2a[s] = r0
        idx2b[s] = jnp.minimum(r0 + 1, jnp.full((16,), ROWS3 - 1, jnp.int32))
        return 0

    lax.fori_loop(0, OPAD // 16, pairidx, 0)
    ha = pltpu.make_async_copy(kp.at[b].at[idx2a], rows2a, semG)
    ha.start()
    hb2 = pltpu.make_async_copy(kp.at[b].at[idx2b], rows2b, semB)
    hb2.start()
    ha.wait()
    hb2.wait()

    def xform(p, _):
        s = pl.ds(p * 16, 16)
        rvec = permr_v[s]
        sv = ssel_v[s]
        qv = plsc.load_gather(qidx_st, [rvec])
        e51 = qv * D
        off = e51 - lax.shift_left(lax.shift_right_logical(e51, 7), 7) + permc_v[s]
        hi = off >= 128
        va = plsc.load_gather(rows2a, [rvec, jnp.minimum(off, 127)])
        vb = plsc.load_gather(rows2b, [rvec, jnp.maximum(off - 128, 0)])
        vals = jnp.where(hi, vb, va)
        scale = jnp.where(sv == 0.0, wsp, jnp.where(sv == 1.0, hsp, jnp.ones((16,), jnp.float32)))
        kpo_st[s] = vals * scale
        return 0

    with scope("ph_xform"):
        lax.fori_loop(0, FLAT // 16, xform, 0)

    pltpu.sync_copy(scores_st, scores_o.at[b])
    pltpu.sync_copy(labels_st, labels_o.at[b])
    pltpu.sync_copy(kpo_st, kpo_o.at[b])


@jax.jit
def _run(pred_logits, pred_keypoints, target_sizes):
    logits2d = pred_logits.reshape(B, N)
    kp = pred_keypoints.reshape(B, ROWS3, 128)
    ts_pad = jnp.pad(target_sizes, ((0, 0), (0, 14)))
    permr = jnp.asarray(_PERM_ROW)
    permc = jnp.asarray(_PERM_COL)
    ssel = jnp.asarray(_SSEL_F)
    mesh = plsc.VectorSubcoreMesh(core_axis_name="c", subcore_axis_name="s",
                                  num_cores=2, num_subcores=16)
    fn = pl.kernel(
        _sc_body,
        out_type=(
            jax.ShapeDtypeStruct((B, OPAD), jnp.float32),
            jax.ShapeDtypeStruct((B, OPAD), jnp.int32),
            jax.ShapeDtypeStruct((B, FLAT), jnp.float32),
        ),
        mesh=mesh,
        compiler_params=pltpu.CompilerParams(needs_layout_passes=False),
        scratch_types=[
            pltpu.VMEM((CH,), jnp.float32),
            pltpu.VMEM((CH,), jnp.float32),
            pltpu.VMEM((CAP,), jnp.int32),
            pltpu.VMEM((CAP,), jnp.int32),
            pltpu.VMEM((CH // GV,), jnp.int32),
            pltpu.VMEM((WPAD,), jnp.int32),
            pltpu.VMEM((WPAD,), jnp.int32),
            pltpu.VMEM((OPAD,), jnp.float32),
            pltpu.VMEM((OPAD,), jnp.int32),
            pltpu.VMEM((OPAD,), jnp.int32),
            pltpu.VMEM((OPAD,), jnp.int32),
            pltpu.VMEM((OPAD,), jnp.int32),
            pltpu.VMEM((OPAD, 128), jnp.float32),
            pltpu.VMEM((OPAD, 128), jnp.float32),
            pltpu.VMEM((FLAT,), jnp.float32),
            pltpu.VMEM((FLAT,), jnp.int32),
            pltpu.VMEM((FLAT,), jnp.int32),
            pltpu.VMEM((FLAT,), jnp.float32),
            pltpu.VMEM((16,), jnp.float32),
            pltpu.SemaphoreType.DMA,
            pltpu.SemaphoreType.DMA,
            pltpu.SemaphoreType.DMA,
        ],
    )
    scores_p, labels_p, kpo_f = fn(logits2d, kp, ts_pad, permr, permc, ssel)
    scores = scores_p[:, :K]
    labels = labels_p[:, :K]
    kpo = kpo_f.reshape(B, OPAD, D)[:, :K, :]
    return scores, labels, kpo


def kernel(pred_logits, pred_keypoints, target_sizes):
    return _run(pred_logits, pred_keypoints, target_sizes)


# speedup vs baseline: 1.0007x; 1.0007x over previous
import functools

import numpy as np
import jax
import jax.numpy as jnp
from jax import lax
from jax.experimental import pallas as pl
from jax.experimental.pallas import tpu as pltpu
from jax.experimental.pallas import tpu_sc as plsc

B, Q, C = 32, 2048, 80
N = Q * C
K = 100
NBP = 17
D = 3 * NBP
CH = 8192
NCH = N // CH
GV = 32
CAP = 2048
NVCAP = CAP // 16
WPAD = 112
OPAD = 128
FLAT = OPAD * D
ROWS3 = Q * D // 128
MINI = np.int32(-2147483648)

_PERM = np.zeros(D, np.int32)
_SSEL = np.zeros(D, np.float32)
for _t in range(NBP):
    _PERM[3 * _t] = 2 * _t
    _PERM[3 * _t + 1] = 2 * _t + 1
    _PERM[3 * _t + 2] = 2 * NBP + _t
    _SSEL[3 * _t] = 0.0
    _SSEL[3 * _t + 1] = 1.0
    _SSEL[3 * _t + 2] = 2.0
_PERM_ROW = np.repeat(np.arange(OPAD, dtype=np.int32), D)
_PERM_COL = np.tile(_PERM, OPAD)
_SSEL_F = np.tile(_SSEL, OPAD)


def _skey(x):
    b = lax.bitcast_convert_type(x, jnp.int32)
    return b ^ lax.shift_right_logical(lax.shift_right_arithmetic(b, 31), 1)


def _fval(k):
    b = k ^ lax.shift_right_logical(lax.shift_right_arithmetic(k, 31), 1)
    return lax.bitcast_convert_type(b, jnp.float32)


def _popc(m):
    return jnp.sum(jnp.where(m, 1, 0))


def _sc_body(logits, kp, ts, permr, permc, ssel,
             scores_o, labels_o, kpo_o,
             chunkA, chunkB, keybuf, idxbuf, summ, wkey, widx,
             scores_st, labels_st, qidx_st, idx2a, idx2b, rows2a, rows2b, kpo_st,
             permr_v, permc_v, ssel_v, ts_v,
             semA, semB, semG):
    b = lax.axis_index("c") * 16 + lax.axis_index("s")
    iota = lax.iota(jnp.int32, 16)
    zeros16 = jnp.zeros((16,), jnp.int32)

    def count_ge(cand_s):
        cb = jnp.full((16,), cand_s, jnp.int32)

        def step(k, acc):
            uk = keybuf[pl.ds(k * 16, 16)]
            return acc + jnp.where(uk >= cb, 1, 0)

        return jnp.sum(lax.fori_loop(0, NVCAP, step, zeros16))

    def descent(limit):
        def cond(st):
            bit, _, c = st
            return (bit >= 0) & (c > limit)

        def body(st):
            bit, tb, c = st
            cand_b = tb | lax.shift_left(jnp.int32(1), bit)
            c2 = count_ge(cand_b ^ MINI)
            ok = c2 >= K
            return bit - 1, jnp.where(ok, cand_b, tb), jnp.where(ok, c2, c)

        _, tb, c = lax.while_loop(cond, body, (jnp.int32(31), jnp.int32(0), jnp.int32(2**30)))
        return tb ^ MINI, c

    def compact(cnt):
        t_s, nc0 = descent(jnp.int32(112))
        tb = jnp.full((16,), t_s, jnp.int32)

        def step(k, nc):
            uk = keybuf[pl.ds(k * 16, 16)]
            ii = idxbuf[pl.ds(k * 16, 16)]
            m = uk >= tb
            plsc.store_compressed(keybuf.at[pl.ds(nc, 16)], uk, mask=m)
            plsc.store_compressed(idxbuf.at[pl.ds(nc, 16)], ii, mask=m)
            return nc + _popc(m)

        nc = lax.fori_loop(0, NVCAP, step, jnp.int32(0))
        nc = jnp.minimum(nc, jnp.int32(CAP - 640))

        def zstep(k, _):
            pos = jnp.full((16,), k * 16, jnp.int32) + iota
            uk = keybuf[pl.ds(k * 16, 16)]
            keybuf[pl.ds(k * 16, 16)] = jnp.where(pos >= nc, MINI, uk)
            return 0

        lax.fori_loop(0, NVCAP, zstep, 0)
        return nc, jnp.max(_fval(jnp.full((16,), t_s, jnp.int32)))

    def process(chunk, cbase, carry, g0=0):
        _, thr_in = carry
        thv1 = jnp.full((16,), thr_in, jnp.float32)

        @plsc.parallel_loop(g0, CH // (GV * 16), unroll=1)
        def _pass1(g):
            xs = [chunk[pl.ds(g * (GV * 16) + v * 16, 16)] for v in range(GV)]
            while len(xs) > 1:
                xs = [jnp.maximum(a, b2) for a, b2 in zip(xs[::2], xs[1::2])]
            summ[pl.ds(g * 16, 16)] = jnp.where(xs[0] >= thv1, 1, 0)

        def group(g, st):
            def slow(st2):
                cnt2, thr2 = st2
                thv2 = jnp.full((16,), thr2, jnp.float32)
                for v in range(GV):
                    x = chunk[pl.ds(g * (GV * 16) + v * 16, 16)]
                    m = (x >= thv2) & jnp.full((16,), cnt2 <= CAP - 16, jnp.bool_)
                    idxv = jnp.full((16,), cbase + g * (GV * 16) + v * 16, jnp.int32) + iota
                    plsc.store_compressed(keybuf.at[pl.ds(cnt2, 16)], _skey(x), mask=m)
                    plsc.store_compressed(idxbuf.at[pl.ds(cnt2, 16)], idxv, mask=m)
                    cnt2 = cnt2 + _popc(m)
                return lax.cond(cnt2 > CAP - 640, lambda c: compact(c), lambda c: (c, thr2), cnt2)

            sm = summ[pl.ds(g * 16, 16)]
            return lax.cond(jnp.any(sm > 0), slow, lambda st2: st2, st)

        return lax.fori_loop(g0, CH // (GV * 16), group, carry)

    scope = jax.named_scope
    row = logits.at[b]
    pltpu.make_async_copy(row.at[pl.ds(0, CH)], chunkA, semA).start()

    def init_kb(k, _):
        keybuf[pl.ds(k * 16, 16)] = jnp.full((16,), MINI, jnp.int32)
        return 0

    with scope("ph_init"):
        lax.fori_loop(0, NVCAP, init_kb, 0)
    for s in range(OPAD // 16):
        qidx_st[pl.ds(s * 16, 16)] = zeros16
    pltpu.sync_copy(ts.at[b], ts_v)
    pltpu.sync_copy(permr, permr_v)
    pltpu.sync_copy(permc, permc_v)
    pltpu.sync_copy(ssel, ssel_v)

    with scope("ph_seed"):
        pltpu.make_async_copy(row.at[pl.ds(0, CH)], chunkA, semA).wait()

        def seed(k, _):
            keybuf[pl.ds(k * 16, 16)] = _skey(chunkA[pl.ds(k * 16, 16)])
            idxbuf[pl.ds(k * 16, 16)] = jnp.full((16,), k * 16, jnp.int32) + iota
            return 0

        lax.fori_loop(0, NVCAP, seed, 0)
        carry = compact(jnp.int32(CAP))

    pltpu.make_async_copy(row.at[pl.ds(CH, CH)], chunkB, semB).start()
    with scope("ph_filter"):
        carry = process(chunkA, 0, carry, g0=CAP // (GV * 16))

        def dbl(i, c):
            cB = 2 * i + 1
            with scope("ph_dmaw"):
                pltpu.make_async_copy(row.at[pl.ds(cB * CH, CH)], chunkB, semB).wait()

                @pl.when(cB < NCH - 1)
                def _():
                    pltpu.make_async_copy(row.at[pl.ds((cB + 1) * CH, CH)], chunkA, semA).start()

            with scope("ph_proc"):
                c = process(chunkB, cB * CH, c)

            with scope("ph_dmaw"):
                @pl.when(cB < NCH - 1)
                def _():
                    pltpu.make_async_copy(row.at[pl.ds((cB + 1) * CH, CH)], chunkA, semA).wait()

                @pl.when(i < NCH // 2 - 1)
                def _():
                    pltpu.make_async_copy(row.at[pl.ds((cB + 2) * CH, CH)], chunkB, semB).start()

            with scope("ph_proc"):
                return lax.cond(cB < NCH - 1,
                                lambda c2: process(chunkA, (cB + 1) * CH, c2),
                                lambda c2: c2, c)

        carry = lax.fori_loop(0, NCH // 2, dbl, carry)

    t_s, _ = descent(jnp.int32(K))
    tb = jnp.full((16,), t_s, jnp.int32)

    def gcount(k, acc):
        uk = keybuf[pl.ds(k * 16, 16)]
        return acc + jnp.where(uk > tb, 1, 0)

    g = jnp.sum(lax.fori_loop(0, NVCAP, gcount, zeros16))
    r = jnp.int32(K) - g

    for s in range(WPAD // 16):
        wkey[pl.ds(s * 16, 16)] = jnp.full((16,), MINI, jnp.int32)
        widx[pl.ds(s * 16, 16)] = jnp.full((16,), jnp.int32(2**31 - 1), jnp.int32)

    def sel(k, st):
        wc, e = st
        uk = keybuf[pl.ds(k * 16, 16)]
        ii = idxbuf[pl.ds(k * 16, 16)]
        m_gt = uk > tb
        m_eq = uk == tb
        pref = plsc.cumsum(jnp.where(m_eq, 1, 0))
        keep_eq = m_eq & ((pref + e) <= r)
        m = m_gt | keep_eq
        plsc.store_compressed(wkey.at[pl.ds(wc, 16)], uk, mask=m)
        plsc.store_compressed(widx.at[pl.ds(wc, 16)], ii, mask=m)
        return wc + _popc(m), e + _popc(m_eq)

    lax.fori_loop(0, NVCAP, sel, (jnp.int32(0), jnp.int32(0)))

    tsrow = ts_v[...]
    hsp = jnp.full((16,), jnp.sum(jnp.where(iota == 0, tsrow, 0.0)), jnp.float32)
    wsp = jnp.full((16,), jnp.sum(jnp.where(iota == 1, tsrow, 0.0)), jnp.float32)
    lane0 = iota == 0

    def rank1(i, _):
        isp = jnp.full((16,), i, jnp.int32)
        ki = plsc.load_gather(wkey, [isp])
        ni = plsc.load_gather(widx, [isp])

        def racc(a, acc):
            wk = wkey[pl.ds(a * 16, 16)]
            wi = widx[pl.ds(a * 16, 16)]
            better = (wk > ki) | ((wk == ki) & (wi < ni))
            return acc + jnp.where(better, 1, 0)

        rank = jnp.sum(lax.fori_loop(0, WPAD // 16, racc, zeros16))
        rsp = jnp.full((16,), rank, jnp.int32)
        score = 1.0 / (1.0 + jnp.exp(-_fval(ki)))
        plsc.store_scatter(scores_st, [rsp], score, mask=lane0)
        plsc.store_scatter(labels_st, [rsp], lax.rem(ni, jnp.full((16,), C, jnp.int32)), mask=lane0)
        plsc.store_scatter(qidx_st, [rsp], lax.div(ni, jnp.full((16,), C, jnp.int32)), mask=lane0)
        return 0

    with scope("ph_rank"):
        lax.fori_loop(0, K, rank1, 0)

    def pairidx(j, _):
        s = pl.ds(j * 16, 16)
        q = qidx_st[s]
        r0 = lax.shift_right_logical(q * D, 7)
        idx2a[s] = r0
        idx2b[s] = jnp.minimum(r0 + 1, jnp.full((16,), ROWS3 - 1, jnp.int32))
        return 0

    lax.fori_loop(0, OPAD // 16, pairidx, 0)
    ha = pltpu.make_async_copy(kp.at[b].at[idx2a], rows2a, semG)
    ha.start()
    hb2 = pltpu.make_async_copy(kp.at[b].at[idx2b], rows2b, semB)
    hb2.start()
    ha.wait()
    hb2.wait()

    def xform(p, _):
        s = pl.ds(p * 16, 16)
        rvec = permr_v[s]
        sv = ssel_v[s]
        qv = plsc.load_gather(qidx_st, [rvec])
        e51 = qv * D
        off = e51 - lax.shift_left(lax.shift_right_logical(e51, 7), 7) + permc_v[s]
        hi = off >= 128
        va = plsc.load_gather(rows2a, [rvec, jnp.minimum(off, 127)])
        vb = plsc.load_gather(rows2b, [rvec, jnp.maximum(off - 128, 0)])
        vals = jnp.where(hi, vb, va)
        scale = jnp.where(sv == 0.0, wsp, jnp.where(sv == 1.0, hsp, jnp.ones((16,), jnp.float32)))
        kpo_st[s] = vals * scale
        return 0

    with scope("ph_xform"):
        lax.fori_loop(0, FLAT // 16, xform, 0)

    pltpu.sync_copy(scores_st, scores_o.at[b])
    pltpu.sync_copy(labels_st, labels_o.at[b])
    pltpu.sync_copy(kpo_st, kpo_o.at[b])


@jax.jit
def _run(pred_logits, pred_keypoints, target_sizes):
    logits2d = pred_logits.reshape(B, N)
    kp = pred_keypoints.reshape(B, ROWS3, 128)
    ts_pad = jnp.pad(target_sizes, ((0, 0), (0, 14)))
    permr = jnp.asarray(_PERM_ROW)
    permc = jnp.asarray(_PERM_COL)
    ssel = jnp.asarray(_SSEL_F)
    mesh = plsc.VectorSubcoreMesh(core_axis_name="c", subcore_axis_name="s",
                                  num_cores=2, num_subcores=16)
    fn = pl.kernel(
        _sc_body,
        out_type=(
            jax.ShapeDtypeStruct((B, OPAD), jnp.float32),
            jax.ShapeDtypeStruct((B, OPAD), jnp.int32),
            jax.ShapeDtypeStruct((B, FLAT), jnp.float32),
        ),
        mesh=mesh,
        compiler_params=pltpu.CompilerParams(needs_layout_passes=False),
        scratch_types=[
            pltpu.VMEM((CH,), jnp.float32),
            pltpu.VMEM((CH,), jnp.float32),
            pltpu.VMEM((CAP,), jnp.int32),
            pltpu.VMEM((CAP,), jnp.int32),
            pltpu.VMEM((CH // GV,), jnp.int32),
            pltpu.VMEM((WPAD,), jnp.int32),
            pltpu.VMEM((WPAD,), jnp.int32),
            pltpu.VMEM((OPAD,), jnp.float32),
            pltpu.VMEM((OPAD,), jnp.int32),
            pltpu.VMEM((OPAD,), jnp.int32),
            pltpu.VMEM((OPAD,), jnp.int32),
            pltpu.VMEM((OPAD,), jnp.int32),
            pltpu.VMEM((OPAD, 128), jnp.float32),
            pltpu.VMEM((OPAD, 128), jnp.float32),
            pltpu.VMEM((FLAT,), jnp.float32),
            pltpu.VMEM((FLAT,), jnp.int32),
            pltpu.VMEM((FLAT,), jnp.int32),
            pltpu.VMEM((FLAT,), jnp.float32),
            pltpu.VMEM((16,), jnp.float32),
            pltpu.SemaphoreType.DMA,
            pltpu.SemaphoreType.DMA,
            pltpu.SemaphoreType.DMA,
        ],
    )
    scores_p, labels_p, kpo_f = fn(logits2d, kp, ts_pad, permr, permc, ssel)
    scores = scores_p[:, :K]
    labels = labels_p[:, :K]
    kpo = kpo_f.reshape(B, OPAD, D)[:, :K, :]
    return scores, labels, kpo


def kernel(pred_logits, pred_keypoints, target_sizes):
    return _run(pred_logits, pred_keypoints, target_sizes)
